# Initial kernel scaffold; baseline (speedup 1.0000x reference)
import jax
import jax.numpy as jnp
from jax.experimental import pallas as pl


def kernel(bboxes, targets, num_level_bboxes):
    raise NotImplementedError("write your pallas kernel here")


# speedup vs baseline: 3.1166x; 3.1166x over previous
import functools

import jax
import jax.numpy as jnp
from jax import lax
from jax.experimental import pallas as pl
from jax.experimental.pallas import tpu as pltpu
from jax.experimental.pallas import tpu_sc as plsc

_NC = 2
_NS = 16
_NW = _NC * _NS
_L = 16

_POS_THR = 0.5
_NEG_THR = 0.4
_BIG = 2e9


def _build_sc_call(n_pad, g_pad, per_w, k_unroll):
    n_chunks = per_w // (k_unroll * _L)

    def body(anc_hbm, gt_hbm, inds_hbm, mo_hbm, labs_hbm,
             ax0_v, ay0_v, ax1_v, ay1_v,
             gx0_v, gy0_v, gx1_v, gy1_v, glab_v, garea_v, compact_v,
             oinds_v, omo_v, olabs_v):
        wid = lax.axis_index("s") * _NC + lax.axis_index("c")
        base = wid * per_w

        pltpu.sync_copy(anc_hbm.at[pl.ds(0 * n_pad + base, per_w)], ax0_v)
        pltpu.sync_copy(anc_hbm.at[pl.ds(1 * n_pad + base, per_w)], ay0_v)
        pltpu.sync_copy(anc_hbm.at[pl.ds(2 * n_pad + base, per_w)], ax1_v)
        pltpu.sync_copy(anc_hbm.at[pl.ds(3 * n_pad + base, per_w)], ay1_v)
        pltpu.sync_copy(gt_hbm.at[pl.ds(0 * g_pad, g_pad)], gx0_v)
        pltpu.sync_copy(gt_hbm.at[pl.ds(1 * g_pad, g_pad)], gy0_v)
        pltpu.sync_copy(gt_hbm.at[pl.ds(2 * g_pad, g_pad)], gx1_v)
        pltpu.sync_copy(gt_hbm.at[pl.ds(3 * g_pad, g_pad)], gy1_v)
        pltpu.sync_copy(gt_hbm.at[pl.ds(4 * g_pad, g_pad)], glab_v)

        big_v = jnp.full((_L,), _BIG, jnp.float32)
        neg1f_v = jnp.full((_L,), -1.0, jnp.float32)
        zero_v = jnp.zeros((_L,), jnp.float32)
        eps_v = jnp.full((_L,), 1e-6, jnp.float32)
        pos_v = jnp.full((_L,), _POS_THR, jnp.float32)
        negthr_v = jnp.full((_L,), _NEG_THR, jnp.float32)
        one_iv = jnp.full((_L,), 1, jnp.int32)
        neg1_iv = jnp.full((_L,), -1, jnp.int32)
        zero_iv = jnp.zeros((_L,), jnp.int32)

        off_v = zero_iv - one_iv
        for j in range(g_pad // _L):
            sl = pl.ds(j * _L, _L)
            valid = glab_v[sl] != neg1f_v
            gx0 = jnp.where(valid, gx0_v[sl], big_v)
            gy0 = jnp.where(valid, gy0_v[sl], big_v)
            gx1 = jnp.where(valid, gx1_v[sl], big_v)
            gy1 = jnp.where(valid, gy1_v[sl], big_v)
            gx0_v[sl] = gx0
            gy0_v[sl] = gy0
            gx1_v[sl] = gx1
            gy1_v[sl] = gy1
            garea_v[sl] = (gx1 - gx0) * (gy1 - gy0)
            vi = jnp.where(valid, one_iv, zero_iv)
            cum = jnp.cumsum(vi)
            compact_v[sl] = cum + off_v
            off_v = plsc.load_gather(
                compact_v, [jnp.full((_L,), j * _L + _L - 1, jnp.int32)])
        for oc in range(n_chunks):
            cbase = oc * k_unroll * _L
            anchors = []
            for k in range(k_unroll):
                sl = pl.ds(cbase + k * _L, _L)
                x0 = ax0_v[sl]
                y0 = ay0_v[sl]
                x1 = ax1_v[sl]
                y1 = ay1_v[sl]
                anchors.append((x0, y0, x1, y1, (x1 - x0) * (y1 - y0)))

            def step(j, carry, anchors=anchors):
                bious, bargs = carry
                idx = jnp.full((_L,), j, dtype=jnp.int32)
                gx0 = plsc.load_gather(gx0_v, [idx])
                gy0 = plsc.load_gather(gy0_v, [idx])
                gx1 = plsc.load_gather(gx1_v, [idx])
                gy1 = plsc.load_gather(gy1_v, [idx])
                ga = plsc.load_gather(garea_v, [idx])
                nb = []
                na = []
                for k in range(k_unroll):
                    x0, y0, x1, y1, aa = anchors[k]
                    w = jnp.maximum(jnp.minimum(x1, gx1) - jnp.maximum(x0, gx0), zero_v)
                    h = jnp.maximum(jnp.minimum(y1, gy1) - jnp.maximum(y0, gy0), zero_v)
                    inter = w * h
                    den = ((aa + ga) - inter) + eps_v
                    iou = inter / den
                    upd = iou > bious[k]
                    nb.append(jnp.where(upd, iou, bious[k]))
                    na.append(jnp.where(upd, idx, bargs[k]))
                return (tuple(nb), tuple(na))

            init = (tuple(neg1f_v for _ in range(k_unroll)),
                    tuple(zero_iv for _ in range(k_unroll)))
            bious, bargs = lax.fori_loop(0, g_pad, step, init)

            for k in range(k_unroll):
                sl = pl.ds(cbase + k * _L, _L)
                biou = bious[k]
                barg = bargs[k]
                pos = biou > pos_v
                neg = biou < negthr_v
                cid = plsc.load_gather(compact_v, [barg])
                labi = plsc.load_gather(glab_v, [barg]).astype(jnp.int32)
                oinds_v[sl] = jnp.where(pos, cid + one_iv, jnp.where(neg, zero_iv, neg1_iv))
                omo_v[sl] = biou
                olabs_v[sl] = jnp.where(pos, labi, neg1_iv)

        pltpu.sync_copy(oinds_v, inds_hbm.at[pl.ds(base, per_w)])
        pltpu.sync_copy(omo_v, mo_hbm.at[pl.ds(base, per_w)])
        pltpu.sync_copy(olabs_v, labs_hbm.at[pl.ds(base, per_w)])

    return pl.kernel(
        body,
        out_type=(
            jax.ShapeDtypeStruct((n_pad,), jnp.int32),
            jax.ShapeDtypeStruct((n_pad,), jnp.float32),
            jax.ShapeDtypeStruct((n_pad,), jnp.int32),
        ),
        mesh=plsc.VectorSubcoreMesh(
            core_axis_name="c", subcore_axis_name="s",
            num_cores=_NC, num_subcores=_NS),
        compiler_params=pltpu.CompilerParams(needs_layout_passes=False),
        scratch_types=[
            pltpu.VMEM((per_w,), jnp.float32),
            pltpu.VMEM((per_w,), jnp.float32),
            pltpu.VMEM((per_w,), jnp.float32),
            pltpu.VMEM((per_w,), jnp.float32),
            pltpu.VMEM((g_pad,), jnp.float32),
            pltpu.VMEM((g_pad,), jnp.float32),
            pltpu.VMEM((g_pad,), jnp.float32),
            pltpu.VMEM((g_pad,), jnp.float32),
            pltpu.VMEM((g_pad,), jnp.float32),
            pltpu.VMEM((g_pad,), jnp.float32),
            pltpu.VMEM((g_pad,), jnp.int32),
            pltpu.VMEM((per_w,), jnp.int32),
            pltpu.VMEM((per_w,), jnp.float32),
            pltpu.VMEM((per_w,), jnp.int32),
        ],
    )


def kernel(bboxes, targets, num_level_bboxes):
    n = bboxes.shape[0]
    g = targets.shape[0]
    per_w = -(-n // (_NW * _L)) * _L
    n_pad = per_w * _NW
    g_pad = -(-g // _L) * _L

    anc = jnp.pad(bboxes, ((0, n_pad - n), (0, 0)))
    anc_flat = anc.T.reshape(-1)
    tgt = jnp.pad(targets, ((0, g_pad - g), (0, 0)),
                  constant_values=-1.0)
    gt_flat = tgt.T.reshape(-1)

    call = _build_sc_call(n_pad, g_pad, per_w, k_unroll=4)
    inds, mo, labs = call(anc_flat, gt_flat)
    return (inds[:n].astype(jnp.int64),
            mo[:n],
            labs[:n].astype(jnp.int64))
